# Initial kernel scaffold; baseline (speedup 1.0000x reference)
import jax
import jax.numpy as jnp
from jax.experimental import pallas as pl


def kernel(classes, bboxes, anchors):
    raise NotImplementedError("write your pallas kernel here")


# speedup vs baseline: 188.8245x; 188.8245x over previous
import jax
import jax.numpy as jnp
import numpy as np
from jax.experimental import pallas as pl
from jax.experimental.pallas import tpu as pltpu

_BBOX_STD = np.array([0.1, 0.1, 0.2, 0.2], dtype=np.float32)
_PRE = 6000
_NPAD = 6144
_BLK = 512
_NBLK = _NPAD // _BLK
_COUNT = 1000
_THR = 0.7


def _nms_body(rows_ref, cols_ref, keep_ref, mcol_ref):
    rows = rows_ref[0]

    for i in range(_NBLK):
        c0 = i * _BLK
        y1c = rows[0:1, c0:c0 + _BLK]
        x1c = rows[1:2, c0:c0 + _BLK]
        y2c = rows[2:3, c0:c0 + _BLK]
        x2c = rows[3:4, c0:c0 + _BLK]
        area_c = jnp.maximum(y2c - y1c, 0.0) * jnp.maximum(x2c - x1c, 0.0)

        def cross_body(j, smax):
            pb = mcol_ref[pl.ds(j * _BLK, _BLK), :]
            y1p, x1p = pb[:, 0:1], pb[:, 1:2]
            y2p, x2p = pb[:, 2:3], pb[:, 3:4]
            area_p = jnp.maximum(y2p - y1p, 0.0) * jnp.maximum(x2p - x1p, 0.0)
            yy1 = jnp.maximum(y1p, y1c)
            xx1 = jnp.maximum(x1p, x1c)
            yy2 = jnp.minimum(y2p, y2c)
            xx2 = jnp.minimum(x2p, x2c)
            inter = jnp.maximum(yy2 - yy1, 0.0) * jnp.maximum(xx2 - xx1, 0.0)
            iou = inter / (area_p + area_c - inter + 1e-8)
            hit = jnp.max(jnp.where(iou > _THR, 1.0, 0.0), axis=0, keepdims=True)
            return jnp.maximum(smax, hit)

        scross = jnp.zeros((1, _BLK), jnp.float32)
        if i > 0:
            scross = jax.lax.fori_loop(0, i, cross_body, scross)
        base = 1.0 - scross

        sb = cols_ref[0, c0:c0 + _BLK, :]
        y1p, x1p = sb[:, 0:1], sb[:, 1:2]
        y2p, x2p = sb[:, 2:3], sb[:, 3:4]
        area_p = jnp.maximum(y2p - y1p, 0.0) * jnp.maximum(x2p - x1p, 0.0)
        yy1 = jnp.maximum(y1p, y1c)
        xx1 = jnp.maximum(x1p, x1c)
        yy2 = jnp.minimum(y2p, y2c)
        xx2 = jnp.minimum(x2p, x2c)
        inter = jnp.maximum(yy2 - yy1, 0.0) * jnp.maximum(xx2 - xx1, 0.0)
        iou = inter / (area_p + area_c - inter + 1e-8)
        pio = jax.lax.broadcasted_iota(jnp.int32, (_BLK, _BLK), 0)
        cio = jax.lax.broadcasted_iota(jnp.int32, (_BLK, _BLK), 1)
        m_sup = jnp.where((iou > _THR) & (pio < cio), 1.0, 0.0)

        def fix_cond(st):
            return st[1]

        def fix_body(st):
            alive, _ = st
            s = jnp.dot(alive, m_sup, preferred_element_type=jnp.float32)
            new_alive = jnp.where(s > 0.5, 0.0, base)
            return new_alive, jnp.any(new_alive != alive)

        s0 = jnp.dot(base, m_sup, preferred_element_type=jnp.float32)
        alive1 = jnp.where(s0 > 0.5, 0.0, base)
        alive, _ = jax.lax.while_loop(
            fix_cond, fix_body, (alive1, jnp.any(alive1 != base)))

        keep_ref[0, 0:1, c0:c0 + _BLK] = alive

        ident = jnp.where(pio == cio, 1.0, 0.0)
        alive_col = jnp.sum(ident * alive, axis=1, keepdims=True)
        mcol_ref[c0:c0 + _BLK, :] = sb * alive_col


def _nms_keep(boxes_rows, boxes_cols, interpret=False):
    return pl.pallas_call(
        _nms_body,
        grid=(boxes_rows.shape[0],),
        in_specs=[
            pl.BlockSpec((1, 4, _NPAD), lambda b: (b, 0, 0)),
            pl.BlockSpec((1, _NPAD, 4), lambda b: (b, 0, 0)),
        ],
        out_specs=pl.BlockSpec((1, 1, _NPAD), lambda b: (b, 0, 0)),
        out_shape=jax.ShapeDtypeStruct(
            (boxes_rows.shape[0], 1, _NPAD), jnp.float32),
        scratch_shapes=[pltpu.VMEM((_NPAD, 4), jnp.float32)],
        interpret=interpret,
    )(boxes_rows, boxes_cols)


def _apply_deltas_clip(anchors_k, deltas_k):
    height = anchors_k[..., 2] - anchors_k[..., 0]
    width = anchors_k[..., 3] - anchors_k[..., 1]
    center_y = anchors_k[..., 0] + 0.5 * height
    center_x = anchors_k[..., 1] + 0.5 * width
    center_y = center_y + deltas_k[..., 0] * height
    center_x = center_x + deltas_k[..., 1] * width
    height = height * jnp.exp(deltas_k[..., 2])
    width = width * jnp.exp(deltas_k[..., 3])
    y1 = center_y - 0.5 * height
    x1 = center_x - 0.5 * width
    y2 = y1 + height
    x2 = x1 + width
    boxes = jnp.stack([y1, x1, y2, x2], axis=-1)
    return jnp.clip(boxes, 0.0, 1.0)


def kernel(classes, bboxes, anchors):
    scores = classes[:, :, 1]
    deltas = bboxes * jnp.asarray(_BBOX_STD).reshape(1, 1, 4)
    _, ix = jax.lax.top_k(scores, _PRE)
    deltas_k = jnp.take_along_axis(deltas, ix[..., None], axis=1)
    anchors_k = jnp.take_along_axis(anchors, ix[..., None], axis=1)
    boxes = _apply_deltas_clip(anchors_k, deltas_k)

    boxes_p = jnp.pad(boxes, ((0, 0), (0, _NPAD - _PRE), (0, 0)))
    keep_f = _nms_keep(boxes_p.transpose(0, 2, 1), boxes_p)
    keep = keep_f[:, 0, :_PRE] > 0.5

    order = jnp.argsort(jnp.logical_not(keep).astype(jnp.int32), axis=1)
    sel = order[:, :_COUNT]
    valid = jnp.take_along_axis(keep, sel, axis=1).astype(boxes.dtype)
    proposals = jnp.take_along_axis(boxes, sel[..., None], axis=1) * valid[..., None]
    return proposals


# speedup vs baseline: 199.4062x; 1.0560x over previous
import jax
import jax.numpy as jnp
import numpy as np
from jax.experimental import pallas as pl
from jax.experimental.pallas import tpu as pltpu

_BBOX_STD = np.array([0.1, 0.1, 0.2, 0.2], dtype=np.float32)
_PRE = 6000
_NPAD = 6144
_BLK = 512
_NBLK = _NPAD // _BLK
_COUNT = 1000
_OUTPAD = 1024
_THR = 0.7


def _nms_body(rows_ref, cols_ref, out_ref, mcol_ref, keep_ref):
    rows = rows_ref[0]

    for i in range(_NBLK):
        c0 = i * _BLK
        y1c = rows[0:1, c0:c0 + _BLK]
        x1c = rows[1:2, c0:c0 + _BLK]
        y2c = rows[2:3, c0:c0 + _BLK]
        x2c = rows[3:4, c0:c0 + _BLK]
        area_c = jnp.maximum(y2c - y1c, 0.0) * jnp.maximum(x2c - x1c, 0.0)

        def cross_body(j, smax):
            pb = mcol_ref[pl.ds(j * _BLK, _BLK), :]
            y1p, x1p = pb[:, 0:1], pb[:, 1:2]
            y2p, x2p = pb[:, 2:3], pb[:, 3:4]
            area_p = jnp.maximum(y2p - y1p, 0.0) * jnp.maximum(x2p - x1p, 0.0)
            yy1 = jnp.maximum(y1p, y1c)
            xx1 = jnp.maximum(x1p, x1c)
            yy2 = jnp.minimum(y2p, y2c)
            xx2 = jnp.minimum(x2p, x2c)
            inter = jnp.maximum(yy2 - yy1, 0.0) * jnp.maximum(xx2 - xx1, 0.0)
            iou = inter / (area_p + area_c - inter + 1e-8)
            hit = jnp.max(jnp.where(iou > _THR, 1.0, 0.0), axis=0, keepdims=True)
            return jnp.maximum(smax, hit)

        scross = jnp.zeros((1, _BLK), jnp.float32)
        if i > 0:
            scross = jax.lax.fori_loop(0, i, cross_body, scross)
        base = 1.0 - scross

        sb = cols_ref[0, c0:c0 + _BLK, :]
        y1p, x1p = sb[:, 0:1], sb[:, 1:2]
        y2p, x2p = sb[:, 2:3], sb[:, 3:4]
        area_p = jnp.maximum(y2p - y1p, 0.0) * jnp.maximum(x2p - x1p, 0.0)
        yy1 = jnp.maximum(y1p, y1c)
        xx1 = jnp.maximum(x1p, x1c)
        yy2 = jnp.minimum(y2p, y2c)
        xx2 = jnp.minimum(x2p, x2c)
        inter = jnp.maximum(yy2 - yy1, 0.0) * jnp.maximum(xx2 - xx1, 0.0)
        iou = inter / (area_p + area_c - inter + 1e-8)
        pio = jax.lax.broadcasted_iota(jnp.int32, (_BLK, _BLK), 0)
        cio = jax.lax.broadcasted_iota(jnp.int32, (_BLK, _BLK), 1)
        m_sup = jnp.where((iou > _THR) & (pio < cio), 1.0, 0.0)

        def fix_cond(st):
            return st[1]

        def fix_body(st):
            alive, _ = st
            s = jnp.dot(alive, m_sup, preferred_element_type=jnp.float32)
            new_alive = jnp.where(s > 0.5, 0.0, base)
            return new_alive, jnp.any(new_alive != alive)

        s0 = jnp.dot(base, m_sup, preferred_element_type=jnp.float32)
        alive1 = jnp.where(s0 > 0.5, 0.0, base)
        alive, _ = jax.lax.while_loop(
            fix_cond, fix_body, (alive1, jnp.any(alive1 != base)))

        keep_ref[0:1, c0:c0 + _BLK] = alive

        ident = jnp.where(pio == cio, 1.0, 0.0)
        alive_col = jnp.sum(ident * alive, axis=1, keepdims=True)
        mcol_ref[c0:c0 + _BLK, :] = sb * alive_col

    pio2 = jax.lax.broadcasted_iota(jnp.int32, (_BLK, _BLK), 0)
    cio2 = jax.lax.broadcasted_iota(jnp.int32, (_BLK, _BLK), 1)
    ut = jnp.where(pio2 <= cio2, 1.0, 0.0)
    oio = jax.lax.broadcasted_iota(jnp.int32, (_OUTPAD, _BLK), 0)

    def compact_body(cb, carry):
        run = carry
        kb = keep_ref[0:1, pl.ds(cb * _BLK, _BLK)]
        cum = jnp.dot(kb, ut, preferred_element_type=jnp.float32) + run
        kr = jnp.where(kb > 0.5, cum - 1.0, -5.0).astype(jnp.int32)
        p3 = jnp.where(kr == oio, 1.0, 0.0)
        src = cols_ref[0, pl.ds(cb * _BLK, _BLK), :]
        out_ref[0] = out_ref[0] + jnp.dot(
            p3, src, preferred_element_type=jnp.float32)
        return run + jnp.sum(kb)

    out_ref[...] = jnp.zeros_like(out_ref)
    jax.lax.fori_loop(0, _NBLK, compact_body, jnp.float32(0.0))


def _nms_keep(boxes_rows, boxes_cols, interpret=False):
    return pl.pallas_call(
        _nms_body,
        grid=(boxes_rows.shape[0],),
        in_specs=[
            pl.BlockSpec((1, 4, _NPAD), lambda b: (b, 0, 0)),
            pl.BlockSpec((1, _NPAD, 4), lambda b: (b, 0, 0)),
        ],
        out_specs=pl.BlockSpec((1, _OUTPAD, 4), lambda b: (b, 0, 0)),
        out_shape=jax.ShapeDtypeStruct(
            (boxes_rows.shape[0], _OUTPAD, 4), jnp.float32),
        scratch_shapes=[
            pltpu.VMEM((_NPAD, 4), jnp.float32),
            pltpu.VMEM((1, _NPAD), jnp.float32),
        ],
        interpret=interpret,
    )(boxes_rows, boxes_cols)


def _apply_deltas_clip(anchors_k, deltas_k):
    height = anchors_k[..., 2] - anchors_k[..., 0]
    width = anchors_k[..., 3] - anchors_k[..., 1]
    center_y = anchors_k[..., 0] + 0.5 * height
    center_x = anchors_k[..., 1] + 0.5 * width
    center_y = center_y + deltas_k[..., 0] * height
    center_x = center_x + deltas_k[..., 1] * width
    height = height * jnp.exp(deltas_k[..., 2])
    width = width * jnp.exp(deltas_k[..., 3])
    y1 = center_y - 0.5 * height
    x1 = center_x - 0.5 * width
    y2 = y1 + height
    x2 = x1 + width
    boxes = jnp.stack([y1, x1, y2, x2], axis=-1)
    return jnp.clip(boxes, 0.0, 1.0)


def kernel(classes, bboxes, anchors):
    scores = classes[:, :, 1]
    deltas = bboxes * jnp.asarray(_BBOX_STD).reshape(1, 1, 4)
    _, ix = jax.lax.top_k(scores, _PRE)
    deltas_k = jnp.take_along_axis(deltas, ix[..., None], axis=1)
    anchors_k = jnp.take_along_axis(anchors, ix[..., None], axis=1)
    boxes = _apply_deltas_clip(anchors_k, deltas_k)

    boxes_p = jnp.pad(boxes, ((0, 0), (0, _NPAD - _PRE), (0, 0)))
    out = _nms_keep(boxes_p.transpose(0, 2, 1), boxes_p)
    return out[:, :_COUNT, :]
